# baseline (speedup 1.0000x reference)
import functools

import jax
import jax.numpy as jnp
from jax import lax
from jax.experimental import pallas as pl
from jax.experimental.pallas import tpu as pltpu
from jax.experimental.pallas import tpu_sc as plsc

N = 49152
H = 64
NUM_MOD = 3
CHUNK = N // NUM_MOD

NC = 2
NS = 16
NW = NC * NS
PER_W = N // NW
CH = 128
NCH = PER_W // CH

NP = N // 2
PRB = 2048
_NB = (CHUNK // 2) // PRB


def _mm_body(x_ref, w_ref, o_ref):
    o_ref[...] = jnp.dot(x_ref[...], w_ref[0], preferred_element_type=jnp.float32)


def _expert_matmul(xp, wd):
    return pl.pallas_call(
        _mm_body,
        grid=(NUM_MOD, _NB),
        in_specs=[
            pl.BlockSpec((PRB, 2 * H), lambda e, b: (e * _NB + b, 0)),
            pl.BlockSpec((1, 2 * H, 2 * H), lambda e, b: (e, 0, 0)),
        ],
        out_specs=pl.BlockSpec((PRB, 2 * H), lambda e, b: (e * _NB + b, 0)),
        out_shape=jax.ShapeDtypeStruct((NP, 2 * H), jnp.float32),
    )(xp, wd)


_sc_mesh = plsc.VectorSubcoreMesh(core_axis_name="c", subcore_axis_name="s")


@functools.partial(
    pl.kernel,
    mesh=_sc_mesh,
    compiler_params=pltpu.CompilerParams(use_tc_tiling_on_sc=False),
    out_type=jax.ShapeDtypeStruct((N, H), jnp.float32),
    scratch_types=[
        pltpu.VMEM((PER_W,), jnp.int32),
        pltpu.VMEM((PER_W,), jnp.int32),
        pltpu.VMEM((PER_W, H), jnp.float32),
        pltpu.SemaphoreType.DMA,
        pltpu.SemaphoreType.DMA,
    ],
)
def _sc_permute(perm_hbm, inv_hbm, y_hbm, out_hbm, perm_v, idx_v, rows_v,
                sem_idx, sem_rows):
    wid = lax.axis_index("s") * NC + lax.axis_index("c")
    pltpu.sync_copy(perm_hbm.at[pl.ds(wid * PER_W, PER_W)], perm_v)
    idx_copies = [
        pltpu.async_copy(inv_hbm.at[perm_v.at[pl.ds(j * CH, CH)]],
                         idx_v.at[pl.ds(j * CH, CH)], sem_idx)
        for j in range(NCH)
    ]
    for c in idx_copies:
        c.wait()
    row_copies = [
        pltpu.async_copy(y_hbm.at[idx_v.at[pl.ds(j * CH, CH)]],
                         rows_v.at[pl.ds(j * CH, CH)], sem_rows)
        for j in range(NCH)
    ]
    for c in row_copies:
        c.wait()
    pltpu.sync_copy(rows_v, out_hbm.at[pl.ds(wid * PER_W, PER_W)])


def _pair_blockdiag(w):
    z = jnp.zeros((H, H), jnp.float32)
    wt = w.T
    return jnp.concatenate(
        [jnp.concatenate([wt, z], axis=1), jnp.concatenate([z, wt], axis=1)],
        axis=0)


def kernel(x, permute_mapping, inv_permute_mapping, W0, W1, W2):
    wd = jnp.stack([_pair_blockdiag(W0), _pair_blockdiag(W1),
                    _pair_blockdiag(W2)])
    xp = x.reshape(NP, 2 * H)
    yp = _expert_matmul(xp, wd)
    y = yp.reshape(N, H)
    return _sc_permute(permute_mapping, inv_permute_mapping, y)


# speedup vs baseline: 1.3384x; 1.3384x over previous
import functools

import jax
import jax.numpy as jnp
from jax import lax
from jax.experimental import pallas as pl
from jax.experimental.pallas import tpu as pltpu
from jax.experimental.pallas import tpu_sc as plsc

N = 49152
H = 64
NUM_MOD = 3
CHUNK = N // NUM_MOD

NC = 2
NS = 16
NW = NC * NS
PER_W = N // NW
CH = 128
NCH = PER_W // CH

NP = N // 2
PRB = 2048
_NB = (CHUNK // 2) // PRB


_NBT = NP // PRB


def _mm_body(xta_ref, xtb_ref, wa_ref, wb_ref, o_ref):
    ya = jax.lax.dot_general(
        xta_ref[...], wa_ref[0], (((0,), (0,)), ((), ())),
        preferred_element_type=jnp.float32)
    yb = jax.lax.dot_general(
        xtb_ref[...], wb_ref[0], (((0,), (0,)), ((), ())),
        preferred_element_type=jnp.float32)
    o_ref[:, 0:H] = ya
    o_ref[:, H:2 * H] = yb


def _expert_matmul(xt, wt):
    return pl.pallas_call(
        _mm_body,
        grid=(_NBT,),
        in_specs=[
            pl.BlockSpec((H, PRB), lambda b: (0, b)),
            pl.BlockSpec((H, PRB), lambda b: (0, b + _NBT)),
            pl.BlockSpec((1, H, H), lambda b: (b * PRB // CHUNK, 0, 0)),
            pl.BlockSpec((1, H, H), lambda b: ((NP + b * PRB) // CHUNK, 0, 0)),
        ],
        out_specs=pl.BlockSpec((PRB, 2 * H), lambda b: (b, 0)),
        out_shape=jax.ShapeDtypeStruct((NP, 2 * H), jnp.float32),
    )(xt, xt, wt, wt)


_sc_mesh = plsc.VectorSubcoreMesh(core_axis_name="c", subcore_axis_name="s")


@functools.partial(
    pl.kernel,
    mesh=_sc_mesh,
    compiler_params=pltpu.CompilerParams(use_tc_tiling_on_sc=False),
    out_type=jax.ShapeDtypeStruct((N, H), jnp.float32),
    scratch_types=[
        pltpu.VMEM((PER_W,), jnp.int32),
        pltpu.VMEM((PER_W,), jnp.int32),
        pltpu.VMEM((PER_W, H), jnp.float32),
        pltpu.SemaphoreType.DMA,
        pltpu.SemaphoreType.DMA,
    ],
)
def _sc_permute(perm_hbm, inv_hbm, y_hbm, out_hbm, perm_v, idx_v, rows_v,
                sem_idx, sem_rows):
    wid = lax.axis_index("s") * NC + lax.axis_index("c")
    pltpu.sync_copy(perm_hbm.at[pl.ds(wid * PER_W, PER_W)], perm_v)
    idx_copies = [
        pltpu.async_copy(inv_hbm.at[perm_v.at[pl.ds(j * CH, CH)]],
                         idx_v.at[pl.ds(j * CH, CH)], sem_idx)
        for j in range(NCH)
    ]
    for c in idx_copies:
        c.wait()

    def _remap(k, carry):
        v = idx_v[pl.ds(k * 16, 16)]
        idx_v[pl.ds(k * 16, 16)] = jnp.where(v >= NP, 2 * v - (2 * NP - 1),
                                             2 * v)
        return carry

    lax.fori_loop(0, PER_W // 16, _remap, 0)
    row_copies = [
        pltpu.async_copy(y_hbm.at[idx_v.at[pl.ds(j * CH, CH)]],
                         rows_v.at[pl.ds(j * CH, CH)], sem_rows)
        for j in range(NCH)
    ]
    for c in row_copies:
        c.wait()
    pltpu.sync_copy(rows_v, out_hbm.at[pl.ds(wid * PER_W, PER_W)])


def kernel(x, permute_mapping, inv_permute_mapping, W0, W1, W2):
    wt = jnp.stack([W0.T, W1.T, W2.T])
    xt = x.T
    yh = _expert_matmul(xt, wt)
    y = yh.reshape(N, H)
    return _sc_permute(permute_mapping, inv_permute_mapping, y)
